# Initial kernel scaffold; baseline (speedup 1.0000x reference)
import jax
import jax.numpy as jnp
from jax.experimental import pallas as pl


def kernel(node_features, segment_ids, W1, b1, W2, b2, W3, b3):
    raise NotImplementedError("write your pallas kernel here")


# speedup vs baseline: 6.8594x; 6.8594x over previous
import jax
import jax.numpy as jnp
from jax import lax
from jax.experimental import pallas as pl

N = 100000
LATENT = 256
HIDDEN = 512
G = 1024

RB = 4000
NB = N // RB


def _seg_body(ids_ref, nodes_ref, sum_ref, cnt_ref):
    i = pl.program_id(0)

    @pl.when(i == 0)
    def _init():
        sum_ref[...] = jnp.zeros_like(sum_ref)
        cnt_ref[...] = jnp.zeros_like(cnt_ref)

    row = lax.broadcasted_iota(jnp.int32, (NB, RB), 0)
    ids_sel = jnp.sum(jnp.where(row == i, ids_ref[...], 0),
                      axis=0, keepdims=True)
    onehot = (ids_sel == lax.broadcasted_iota(jnp.int32, (G, RB), 0)
              ).astype(jnp.float32)
    sum_ref[...] += lax.dot_general(
        onehot, nodes_ref[...],
        dimension_numbers=(((1,), (0,)), ((), ())),
        preferred_element_type=jnp.float32)
    cnt_ref[...] += jnp.sum(onehot, axis=1, keepdims=True)


def _seg_sum(ids2d, nodes):
    return pl.pallas_call(
        _seg_body,
        grid=(NB,),
        in_specs=[
            pl.BlockSpec((NB, RB), lambda i: (0, 0)),
            pl.BlockSpec((RB, LATENT), lambda i: (i, 0)),
        ],
        out_specs=[
            pl.BlockSpec((G, LATENT), lambda i: (0, 0)),
            pl.BlockSpec((G, 1), lambda i: (0, 0)),
        ],
        out_shape=[
            jax.ShapeDtypeStruct((G, LATENT), jnp.float32),
            jax.ShapeDtypeStruct((G, 1), jnp.float32),
        ],
    )(ids2d, nodes)


_LOG2 = 0.6931471805599453


def _ssp(x):
    return jnp.maximum(x, 0.0) + jnp.log(1.0 + jnp.exp(-jnp.abs(x))) - _LOG2


def _mlp_body(sum_ref, cnt_ref, w1_ref, b1_ref, w2_ref, b2_ref, w3_ref,
              b3_ref, out_ref):
    agg = sum_ref[...] / jnp.maximum(cnt_ref[...], 1.0)
    h = jnp.dot(agg, w1_ref[...], preferred_element_type=jnp.float32)
    h = _ssp(h + b1_ref[...])
    h = jnp.dot(h, w2_ref[...], preferred_element_type=jnp.float32)
    h = _ssp(h + b2_ref[...])
    out_ref[...] = (
        jnp.dot(h, w3_ref[...], preferred_element_type=jnp.float32)
        + b3_ref[...])


def _mlp(sums, cnt, W1, b1, W2, b2, W3, b3):
    return pl.pallas_call(
        _mlp_body,
        out_shape=jax.ShapeDtypeStruct((G, 1), jnp.float32),
    )(sums, cnt, W1, b1, W2, b2, W3, b3)


def kernel(node_features, segment_ids, W1, b1, W2, b2, W3, b3):
    ids2d = segment_ids.astype(jnp.int32).reshape(NB, RB)
    sums, cnt = _seg_sum(ids2d, node_features)
    return _mlp(sums, cnt, W1,
                b1.reshape(1, HIDDEN), W2, b2.reshape(1, HIDDEN),
                W3, b3.reshape(1, 1))
